# baseline (speedup 1.0000x reference)
import jax
import jax.numpy as jnp
from jax.experimental import pallas as pl
from jax.experimental.pallas import tpu as pltpu

_B, _C, _H, _W = 64, 3, 512, 512
_P = 16
_DIM = 96
_DP = 128
_G = _H // _P
_N = _G * _G
_K = _C * _P * _P
_KEEP = _N // 2
_PW = 640


def _fused_kernel(p_ref, w1_ref, b1_ref, w2_ref, b2_ref, tri_ref, eye_ref,
                  p2_ref, o_ref):
    p = p_ref[0]
    h = jax.lax.dot_general(p, w1_ref[...], (((1,), (0,)), ((), ())),
                            preferred_element_type=jnp.float32)
    h = h + b1_ref[...]
    hb = h.astype(jnp.bfloat16).astype(jnp.float32)
    prod = hb * w2_ref[...]
    s = prod
    c = jnp.zeros_like(prod)
    width = _DP // 2
    while width >= 1:
        a_s, b_s = s[:, :width], s[:, width:2 * width]
        a_c, b_c = c[:, :width], c[:, width:2 * width]
        t = a_s + b_s
        bb = t - a_s
        err = (a_s - (t - bb)) + (b_s - bb)
        s = t
        c = (a_c + b_c) + err
        width //= 2
    tot = (s + c) + b2_ref[...]

    row = jax.lax.dot_general(tot, eye_ref[...], (((0,), (0,)), ((), ())),
                              preferred_element_type=jnp.float32,
                              precision=jax.lax.Precision.HIGHEST)

    m = jnp.max(row, axis=1, keepdims=True)
    vrow = jnp.exp(row - m)
    vcol = jnp.exp(tot - m)
    brow = jax.lax.bitcast_convert_type(vrow, jnp.int32) * 2
    bcol = jax.lax.bitcast_convert_type(vcol, jnp.int32) * 2
    above = (brow + tri_ref[...]) > bcol
    ab = above.astype(jnp.bfloat16)
    ones = jnp.ones((_N, 1), jnp.bfloat16)
    rank1 = jax.lax.dot_general(ab, ones, (((1,), (0,)), ((), ())),
                                preferred_element_type=jnp.float32) + 1.0

    hit = (rank1 == p2_ref[...]).astype(jnp.float32)
    idx1 = jax.lax.broadcasted_iota(jnp.int32, (1, _N), 1).astype(
        jnp.float32) + 1.0
    out = jax.lax.dot_general(idx1, hit, (((1,), (0,)), ((), ())),
                              preferred_element_type=jnp.float32,
                              precision=jax.lax.Precision.HIGHEST)
    o_ref[0] = out.astype(jnp.int32)


def kernel(x, W1, b1, W2, b2):
    bf16, f32 = jnp.bfloat16, jnp.float32
    patches = x.astype(bf16).reshape(_B, _C, _G, _P, _G, _P)
    patches = patches.transpose(0, 2, 4, 1, 3, 5).reshape(_B, _N, _K)
    w1m = W1.astype(bf16).reshape(_DIM, _K).T
    w1m = jnp.pad(w1m, ((0, 0), (0, _DP - _DIM)))
    b1r = jnp.pad(b1, (0, _DP - _DIM)).reshape(1, _DP)
    w2r = W2.reshape(1, _DIM).astype(bf16).astype(f32)
    w2r = jnp.pad(w2r, ((0, 0), (0, _DP - _DIM)))
    b2r = b2.reshape(1, 1)
    ii = jnp.arange(_N, dtype=jnp.int32)
    tri = (ii[None, :] < ii[:, None]).astype(jnp.int32)
    eye = jnp.eye(_N, dtype=f32)
    p2 = jnp.broadcast_to(jnp.arange(_PW, dtype=f32)[None, :], (_N, _PW))

    idx = pl.pallas_call(
        _fused_kernel,
        grid=(_B,),
        in_specs=[
            pl.BlockSpec((1, _N, _K), lambda b: (b, 0, 0)),
            pl.BlockSpec((_K, _DP), lambda b: (0, 0)),
            pl.BlockSpec((1, _DP), lambda b: (0, 0)),
            pl.BlockSpec((1, _DP), lambda b: (0, 0)),
            pl.BlockSpec((1, 1), lambda b: (0, 0)),
            pl.BlockSpec((_N, _N), lambda b: (0, 0)),
            pl.BlockSpec((_N, _N), lambda b: (0, 0)),
            pl.BlockSpec((_N, _PW), lambda b: (0, 0)),
        ],
        out_specs=pl.BlockSpec((1, 1, _PW), lambda b: (b, 0, 0)),
        out_shape=jax.ShapeDtypeStruct((_B, 1, _PW), jnp.int32),
    )(patches, w1m, b1r, w2r, b2r, tri, eye, p2)

    return idx[:, 0, : _KEEP + 1].astype(jnp.int64)


# speedup vs baseline: 1.1357x; 1.1357x over previous
import jax
import jax.numpy as jnp
from jax.experimental import pallas as pl
from jax.experimental.pallas import tpu as pltpu

_B, _C, _H, _W = 64, 3, 512, 512
_P = 16
_DIM = 96
_DP = 128
_G = _H // _P
_N = _G * _G
_K = _C * _P * _P
_KEEP = _N // 2
_PW = 640


def _score_kernel(p_ref, w1_ref, b1_ref, w2_ref, b2_ref, s_ref):
    p = p_ref[0]
    h = jax.lax.dot_general(p, w1_ref[...], (((1,), (0,)), ((), ())),
                            preferred_element_type=jnp.float32)
    h = h + b1_ref[...]
    hb = h.astype(jnp.bfloat16).astype(jnp.float32)
    prod = hb * w2_ref[...]
    s = prod
    c = jnp.zeros_like(prod)
    width = _DP // 2
    while width >= 1:
        a_s, b_s = s[:, :width], s[:, width:2 * width]
        a_c, b_c = c[:, :width], c[:, width:2 * width]
        t = a_s + b_s
        bb = t - a_s
        err = (a_s - (t - bb)) + (b_s - bb)
        s = t
        c = (a_c + b_c) + err
        width //= 2
    tot = (s + c) + b2_ref[...]
    i2 = jax.lax.broadcasted_iota(jnp.int32, (_N, _N), 0)
    j2 = jax.lax.broadcasted_iota(jnp.int32, (_N, _N), 1)
    eye = (i2 == j2).astype(jnp.float32)
    row = jax.lax.dot_general(tot, eye, (((0,), (0,)), ((), ())),
                              preferred_element_type=jnp.float32,
                              precision=jax.lax.Precision.HIGHEST)
    s_ref[0] = row


def _rank_kernel(s_ref, o_ref):
    sr = s_ref[0]
    row = jnp.exp(sr - jnp.max(sr, axis=1, keepdims=True))
    i2 = jax.lax.broadcasted_iota(jnp.int32, (_N, _N), 0)
    j2 = jax.lax.broadcasted_iota(jnp.int32, (_N, _N), 1)
    eye = (i2 == j2).astype(jnp.float32)
    col = jax.lax.dot_general(eye, row, (((1,), (1,)), ((), ())),
                              preferred_element_type=jnp.float32,
                              precision=jax.lax.Precision.HIGHEST)
    vj = row
    vi = col
    above = (vj > vi) | ((vj == vi) & (j2 < i2))
    rank = jnp.sum(above.astype(jnp.float32), axis=1, keepdims=True)
    p2 = jax.lax.broadcasted_iota(jnp.int32, (_N, _PW), 1).astype(jnp.float32)
    hit = rank == (p2 - 1.0)
    idx1 = i2[:, :_PW].astype(jnp.float32) + 1.0
    out = jnp.sum(jnp.where(hit, idx1, 0.0), axis=0, keepdims=True)
    o_ref[0] = out.astype(jnp.int32)


def kernel(x, W1, b1, W2, b2):
    bf16, f32 = jnp.bfloat16, jnp.float32
    patches = x.astype(bf16).reshape(_B, _C, _G, _P, _G, _P)
    patches = patches.transpose(0, 2, 4, 1, 3, 5).reshape(_B, _N, _K)
    w1m = W1.astype(bf16).reshape(_DIM, _K).T
    w1m = jnp.pad(w1m, ((0, 0), (0, _DP - _DIM)))
    b1r = jnp.pad(b1, (0, _DP - _DIM)).reshape(1, _DP)
    w2r = W2.reshape(1, _DIM).astype(bf16).astype(f32)
    w2r = jnp.pad(w2r, ((0, 0), (0, _DP - _DIM)))
    b2r = b2.reshape(1, 1)

    s = pl.pallas_call(
        _score_kernel,
        grid=(_B,),
        in_specs=[
            pl.BlockSpec((1, _N, _K), lambda b: (b, 0, 0)),
            pl.BlockSpec((_K, _DP), lambda b: (0, 0)),
            pl.BlockSpec((1, _DP), lambda b: (0, 0)),
            pl.BlockSpec((1, _DP), lambda b: (0, 0)),
            pl.BlockSpec((1, 1), lambda b: (0, 0)),
        ],
        out_specs=pl.BlockSpec((1, 1, _N), lambda b: (b, 0, 0)),
        out_shape=jax.ShapeDtypeStruct((_B, 1, _N), jnp.float32),
    )(patches, w1m, b1r, w2r, b2r)

    idx = pl.pallas_call(
        _rank_kernel,
        grid=(_B,),
        in_specs=[pl.BlockSpec((1, 1, _N), lambda b: (b, 0, 0))],
        out_specs=pl.BlockSpec((1, 1, _PW), lambda b: (b, 0, 0)),
        out_shape=jax.ShapeDtypeStruct((_B, 1, _PW), jnp.int32),
    )(s)

    return idx[:, 0, : _KEEP + 1].astype(jnp.int64)
